# baseline (speedup 1.0000x reference)
import functools

import jax
import jax.numpy as jnp
from jax import lax
from jax.experimental import pallas as pl
from jax.experimental.pallas import tpu as pltpu
from jax.experimental.pallas import tpu_sc as plsc

NUM_SEGMENTS = 200
COMP = 9
ACCW = 16
NC = 2
NS = 16
CHUNK = 2048
CWORDS = CHUNK * COMP
ACCSZ = NUM_SEGMENTS * ACCW


def _sc_segment_sum(ids_flat, src_flat, bz, hw):
    bpc = bz // NC
    pix_per_tile = hw // NS
    n_chunks = pix_per_tile // CHUNK
    assert pix_per_tile % CHUNK == 0

    mesh = plsc.VectorSubcoreMesh(core_axis_name="c", subcore_axis_name="s")

    @functools.partial(
        pl.kernel,
        out_type=jax.ShapeDtypeStruct((NC, NS, bpc, ACCSZ), jnp.float32),
        mesh=mesh,
        compiler_params=pltpu.CompilerParams(
            use_tc_tiling_on_sc=False, needs_layout_passes=False),
        scratch_types=[
            [pltpu.VMEM((ACCSZ,), jnp.float32) for _ in range(bpc)],
            [pltpu.VMEM((CWORDS,), jnp.float32) for _ in range(2)],
            [pltpu.VMEM((CHUNK,), jnp.int32) for _ in range(2)],
            [pltpu.SemaphoreType.DMA for _ in range(4)],
        ],
    )
    def seg_sum(ids_hbm, src_hbm, out_hbm, accs, flats, idxs, sems):
        c = lax.axis_index("c")
        s = lax.axis_index("s")
        zero16 = jnp.zeros((16,), jnp.float32)

        def zero_body(i, _):
            for acc in accs:
                acc[pl.ds(i * 16, 16)] = zero16
            return 0

        lax.fori_loop(0, ACCSZ // 16, zero_body, 0)

        lane = lax.iota(jnp.int32, 16)
        pix_off = []
        comp_off = []
        for k in range(COMP):
            flatpos = lane + 16 * k
            p = flatpos // COMP
            pix_off.append(p)
            comp_off.append(flatpos - p * COMP)

        chunks = [(lb, i) for lb in range(bpc) for i in range(n_chunks)]

        def start_dma(g):
            lb, i = chunks[g]
            b = c * bpc + lb
            base = (b * hw + s * pix_per_tile + i * CHUNK)
            j = g % 2
            d1 = pltpu.async_copy(
                src_hbm.at[pl.ds(base * COMP, CWORDS)], flats[j], sems[2 * j])
            d2 = pltpu.async_copy(
                ids_hbm.at[pl.ds(base, CHUNK)], idxs[j], sems[2 * j + 1])
            return (d1, d2)

        pending = start_dma(0)
        for g, (lb, i) in enumerate(chunks):
            j = g % 2
            flat_v, idx_v, acc = flats[j], idxs[j], accs[lb]
            cur = pending
            if g + 1 < len(chunks):
                pending = start_dma(g + 1)
            cur[0].wait()
            cur[1].wait()

            def block(ib, _, flat_v=flat_v, idx_v=idx_v, acc=acc):
                for k in range(COMP):
                    ids16 = plsc.load_gather(idx_v, [ib * 16 + pix_off[k]])
                    addr = ids16 * ACCW + comp_off[k]
                    vals = flat_v[pl.ds(ib * 144 + 16 * k, 16)]
                    plsc.addupdate_scatter(acc, [addr], vals)
                return 0

            lax.fori_loop(0, CHUNK // 16, block, 0)

        for lb in range(bpc):
            pltpu.sync_copy(accs[lb], out_hbm.at[c, s, lb])

    return seg_sum(ids_flat, src_flat)


def _tc_finish(partials):
    nc, ns, bpc = partials.shape[:3]

    def body(x_ref, o_ref):
        summed = jnp.sum(x_ref[...], axis=1)
        scaled = summed[:, :, :, :COMP] * jnp.float32(1.1)
        o_ref[...] = scaled.reshape(nc * bpc, NUM_SEGMENTS, COMP)

    return pl.pallas_call(
        body,
        out_shape=jax.ShapeDtypeStruct((nc * bpc, NUM_SEGMENTS, COMP),
                                       jnp.float32),
    )(partials)


def kernel(instance, compsrc, maxinsnum):
    bz, _, h, w = instance.shape
    hw = h * w
    ids_flat = instance.reshape(bz * hw)
    src_flat = compsrc.reshape(bz * hw * COMP)
    partials = _sc_segment_sum(ids_flat, src_flat, bz, hw)
    partials = partials.reshape(NC, NS, bz // NC, NUM_SEGMENTS, ACCW)
    out = _tc_finish(partials)
    return out.reshape(bz, NUM_SEGMENTS, 3, 3)


# speedup vs baseline: 43.7471x; 43.7471x over previous
import functools

import jax
import jax.numpy as jnp
from jax import lax
from jax.experimental import pallas as pl
from jax.experimental.pallas import tpu as pltpu
from jax.experimental.pallas import tpu_sc as plsc

NUM_SEGMENTS = 200
ACC_PAD = 208
COMP = 9
NC = 2
NS = 16


def _sc_segment_sum(ids3, srcp, bz, h, w):
    bpc = bz // NC
    rows = h // NS
    nxv = w // 16

    mesh = plsc.VectorSubcoreMesh(core_axis_name="c", subcore_axis_name="s")

    @functools.partial(
        pl.kernel,
        out_type=jax.ShapeDtypeStruct((NC, NS, bpc, COMP, NUM_SEGMENTS),
                                      jnp.float32),
        mesh=mesh,
        compiler_params=pltpu.CompilerParams(
            use_tc_tiling_on_sc=False, needs_layout_passes=False),
        scratch_types=[
            [[pltpu.VMEM((ACC_PAD,), jnp.float32) for _ in range(COMP)]
             for _ in range(bpc)],
            [pltpu.VMEM((rows, w), jnp.float32) for _ in range(2)],
            [pltpu.VMEM((rows, w), jnp.int32) for _ in range(bpc)],
            [pltpu.SemaphoreType.DMA for _ in range(2)],
            [pltpu.SemaphoreType.DMA for _ in range(bpc)],
        ],
    )
    def seg_sum(ids_hbm, src_hbm, out_hbm, accs, pbufs, ibufs, psems, isems):
        c = lax.axis_index("c")
        s = lax.axis_index("s")
        y0 = s * rows
        zero16 = jnp.zeros((16,), jnp.float32)

        def zero_body(i, _):
            for lb in range(bpc):
                for k in range(COMP):
                    accs[lb][k][pl.ds(i * 16, 16)] = zero16
            return 0

        lax.fori_loop(0, ACC_PAD // 16, zero_body, 0)

        id_dmas = [
            pltpu.async_copy(ids_hbm.at[c * bpc + lb, pl.ds(y0, rows)],
                             ibufs[lb], isems[lb])
            for lb in range(bpc)
        ]

        planes = [(lb, k) for lb in range(bpc) for k in range(COMP)]

        def start_plane(t):
            lb, k = planes[t]
            return pltpu.async_copy(
                src_hbm.at[c * bpc + lb, k, pl.ds(y0, rows)],
                pbufs[t % 2], psems[t % 2])

        pending = start_plane(0)
        for t, (lb, k) in enumerate(planes):
            pbuf, ibuf, acc = pbufs[t % 2], ibufs[lb], accs[lb][k]
            cur = pending
            if t + 1 < len(planes):
                pending = start_plane(t + 1)
            if k == 0:
                id_dmas[lb].wait()
            cur.wait()

            def y_body(y, _, pbuf=pbuf, ibuf=ibuf, acc=acc):
                for xb in range(nxv):
                    ids16 = ibuf[y, pl.ds(xb * 16, 16)]
                    vals = pbuf[y, pl.ds(xb * 16, 16)]
                    plsc.addupdate_scatter(acc, [ids16], vals)
                return 0

            lax.fori_loop(0, rows, y_body, 0)

        for lb in range(bpc):
            for k in range(COMP):
                pltpu.sync_copy(accs[lb][k].at[pl.ds(0, NUM_SEGMENTS)],
                                out_hbm.at[c, s, lb, k])

    return seg_sum(ids3, srcp)


def _tc_finish(partials):
    nc, ns, bpc = partials.shape[:3]

    def body(x_ref, o_ref):
        summed = jnp.sum(x_ref[...], axis=1)
        scaled = summed * jnp.float32(1.1)
        swapped = jnp.swapaxes(scaled, 2, 3)
        o_ref[...] = swapped.reshape(nc * bpc, NUM_SEGMENTS, COMP)

    return pl.pallas_call(
        body,
        out_shape=jax.ShapeDtypeStruct((nc * bpc, NUM_SEGMENTS, COMP),
                                       jnp.float32),
    )(partials)


def kernel(instance, compsrc, maxinsnum):
    bz, _, h, w = instance.shape
    ids3 = instance.reshape(bz, h, w)
    srcp = compsrc.transpose(0, 3, 4, 1, 2).reshape(bz, COMP, h, w)
    partials = _sc_segment_sum(ids3, srcp, bz, h, w)
    out = _tc_finish(partials)
    return out.reshape(bz, NUM_SEGMENTS, 3, 3)


# speedup vs baseline: 53.7862x; 1.2295x over previous
import functools

import jax
import jax.numpy as jnp
from jax import lax
from jax.experimental import pallas as pl
from jax.experimental.pallas import tpu as pltpu
from jax.experimental.pallas import tpu_sc as plsc

NUM_SEGMENTS = 200
ACC_PAD = 208
COMP = 9
NC = 2
NS = 16
SROWS = 5


def _sc_segment_sum(ids3, srcp, bz, h, w):
    bpc = bz // NC
    rows = h // NS
    nst = rows // SROWS
    nxv = w // 16
    assert rows % SROWS == 0

    mesh = plsc.VectorSubcoreMesh(core_axis_name="c", subcore_axis_name="s")

    @functools.partial(
        pl.kernel,
        out_type=jax.ShapeDtypeStruct((NC, NS, bpc, COMP, NUM_SEGMENTS),
                                      jnp.float32),
        mesh=mesh,
        compiler_params=pltpu.CompilerParams(
            use_tc_tiling_on_sc=False, needs_layout_passes=False),
        scratch_types=[
            [[pltpu.VMEM((ACC_PAD,), jnp.float32) for _ in range(COMP)]
             for _ in range(bpc)],
            [[pltpu.VMEM((SROWS, w), jnp.float32) for _ in range(COMP)]
             for _ in range(2)],
            pltpu.VMEM((rows, w), jnp.int32),
            [pltpu.SemaphoreType.DMA for _ in range(2)],
            pltpu.SemaphoreType.DMA,
        ],
    )
    def seg_sum(ids_hbm, src_hbm, out_hbm, accs, pbufs, ibuf, psems, isem):
        c = lax.axis_index("c")
        s = lax.axis_index("s")
        y0 = s * rows
        zero16 = jnp.zeros((16,), jnp.float32)

        def zero_body(i, _):
            for lb in range(bpc):
                for k in range(COMP):
                    accs[lb][k][pl.ds(i * 16, 16)] = zero16
            return 0

        lax.fori_loop(0, ACC_PAD // 16, zero_body, 0)

        stripes = [(lb, st) for lb in range(bpc) for st in range(nst)]

        def start_stripe(t):
            lb, st = stripes[t]
            j = t % 2
            return [
                pltpu.async_copy(
                    src_hbm.at[c * bpc + lb, k,
                               pl.ds(y0 + st * SROWS, SROWS)],
                    pbufs[j][k], psems[j])
                for k in range(COMP)
            ]

        id_dma = pltpu.async_copy(ids_hbm.at[c * bpc + 0, pl.ds(y0, rows)],
                                  ibuf, isem)
        pending = start_stripe(0)
        id_dma.wait()

        for t, (lb, st) in enumerate(stripes):
            j = t % 2
            cur = pending
            if t + 1 < len(stripes):
                pending = start_stripe(t + 1)
            for d in cur:
                d.wait()
            pb = pbufs[j]
            acc_lb = accs[lb]

            def xb_body(xb, yy, pb=pb, acc_lb=acc_lb, st=st):
                ids16 = ibuf[yy + st * SROWS, pl.ds(xb * 16, 16)]
                for k in range(COMP):
                    vals = pb[k][yy, pl.ds(xb * 16, 16)]
                    plsc.addupdate_scatter(acc_lb[k], [ids16], vals)
                return yy

            def y_body(yy, _, xb_body=xb_body):
                lax.fori_loop(0, nxv, xb_body, yy)
                return 0

            lax.fori_loop(0, SROWS, y_body, 0)

            if bpc > 1 and (lb, st) == (0, nst - 1):
                pltpu.async_copy(ids_hbm.at[c * bpc + 1, pl.ds(y0, rows)],
                                 ibuf, isem).wait()

        for lb in range(bpc):
            for k in range(COMP):
                pltpu.sync_copy(accs[lb][k].at[pl.ds(0, NUM_SEGMENTS)],
                                out_hbm.at[c, s, lb, k])

    return seg_sum(ids3, srcp)


def _tc_finish(partials):
    nc, ns, bpc = partials.shape[:3]

    def body(x_ref, o_ref):
        summed = jnp.sum(x_ref[...], axis=1)
        scaled = summed * jnp.float32(1.1)
        swapped = jnp.swapaxes(scaled, 2, 3)
        o_ref[...] = swapped.reshape(nc * bpc, NUM_SEGMENTS, COMP)

    return pl.pallas_call(
        body,
        out_shape=jax.ShapeDtypeStruct((nc * bpc, NUM_SEGMENTS, COMP),
                                       jnp.float32),
    )(partials)


def kernel(instance, compsrc, maxinsnum):
    bz, _, h, w = instance.shape
    ids3 = instance.reshape(bz, h, w)
    srcp = compsrc.transpose(0, 3, 4, 1, 2).reshape(bz, COMP, h, w)
    partials = _sc_segment_sum(ids3, srcp, bz, h, w)
    out = _tc_finish(partials)
    return out.reshape(bz, NUM_SEGMENTS, 3, 3)
